# baseline (speedup 1.0000x reference)
import jax
import jax.numpy as jnp
from jax import lax
from jax.experimental import pallas as pl
from jax.experimental.pallas import tpu as pltpu
from jax.experimental.pallas import tpu_sc as plsc

N = 10000
E = 160000
DIN = 128
DIM = 32
MESS = 128
B = 64

EP = 163840
NW = 32
CHUNK = 128
CPW = EP // NW // CHUNK
RPW = EP // NW
NPAD = 10016
RPT = NPAD // 16
BE = 1280
NBLK = EP // BE

_f32 = jnp.float32


def _sc_mesh():
    return plsc.VectorSubcoreMesh(core_axis_name="c", subcore_axis_name="s")


def _gather_body(table, idx2, xj, idx_v, rows_v, sem):
    c = lax.axis_index("c")
    s = lax.axis_index("s")
    wid = s * 2 + c
    pltpu.sync_copy(idx2.at[pl.ds(wid * CPW, CPW)], idx_v)

    def body(g, carry):
        cps = [
            pltpu.async_copy(table.at[idx_v.at[g * 4 + j]],
                             rows_v.at[pl.ds(j * CHUNK, CHUNK)], sem)
            for j in range(4)
        ]
        for cp in cps:
            cp.wait()
        pltpu.sync_copy(rows_v, xj.at[pl.ds(wid * RPW + g * 4 * CHUNK, 4 * CHUNK)])
        return carry

    lax.fori_loop(0, CPW // 4, body, 0)


def _sc_gather(table, idx2):
    return pl.kernel(
        _gather_body,
        out_type=jax.ShapeDtypeStruct((EP, DIM), _f32),
        mesh=_sc_mesh(),
        scratch_types=[
            pltpu.VMEM((CPW, CHUNK), jnp.int32),
            pltpu.VMEM((4 * CHUNK, DIM), _f32),
            pltpu.SemaphoreType.DMA,
        ],
        compiler_params=pltpu.CompilerParams(use_tc_tiling_on_sc=False),
    )(table, idx2)


def _scatter_body(msg, idx2, zeros_hbm, aggp, idx_v, rows_v, agg_sh):
    c = lax.axis_index("c")
    s = lax.axis_index("s")
    wid = s * 2 + c
    pltpu.sync_copy(zeros_hbm.at[pl.ds(s * RPT, RPT)], agg_sh.at[pl.ds(s * RPT, RPT)])
    plsc.subcore_barrier()
    pltpu.sync_copy(idx2.at[pl.ds(wid * CPW, CPW)], idx_v)

    def body(g, carry):
        pltpu.sync_copy(msg.at[pl.ds(wid * RPW + g * CHUNK, CHUNK)], rows_v)
        pltpu.sync_copy(rows_v, agg_sh.at[idx_v.at[g]], add=True)
        return carry

    lax.fori_loop(0, CPW, body, 0)
    plsc.subcore_barrier()
    pltpu.sync_copy(agg_sh.at[pl.ds(s * RPT, RPT)], aggp.at[c, pl.ds(s * RPT, RPT)])


def _sc_scatter_add(msg, idx2, zeros_hbm):
    return pl.kernel(
        _scatter_body,
        out_type=jax.ShapeDtypeStruct((2, NPAD, DIM), _f32),
        mesh=_sc_mesh(),
        scratch_types=[
            pltpu.VMEM((CPW, CHUNK), jnp.int32),
            pltpu.VMEM((CHUNK, DIM), _f32),
            pltpu.VMEM_SHARED((NPAD, DIM), _f32),
        ],
        compiler_params=pltpu.CompilerParams(use_tc_tiling_on_sc=False),
    )(msg, idx2, zeros_hbm)


def _lin0_body(x_ref, w_ref, b_ref, o_ref):
    o_ref[...] = jax.nn.relu(
        jnp.dot(x_ref[...], w_ref[...], preferred_element_type=_f32) + b_ref[...])


def _he_body(ea_ref, w_ref, b_ref, o_ref):
    o_ref[...] = jax.nn.relu(
        jnp.dot(ea_ref[...], w_ref[...], preferred_element_type=_f32) + b_ref[...])


def _msg_body(xj_ref, he_ref, w_ref, b_ref, o_ref):
    we = jnp.dot(he_ref[...], w_ref[...], preferred_element_type=_f32) + b_ref[...]
    xj = xj_ref[...]
    acc = xj[:, 0:1] * we[:, 0:DIM]
    for d in range(1, DIM):
        acc = acc + xj[:, d:d + 1] * we[:, d * DIM:(d + 1) * DIM]
    o_ref[...] = acc


def _update_body(a0_ref, a1_ref, d0_ref, d1_ref, out_ref, h_ref, root_ref,
                 cb_ref, wir_ref, wiz_ref, win_ref, whr_ref, whz_ref, whn_ref,
                 bir_ref, biz_ref, bin_ref, bhr_ref, bhz_ref, bhn_ref, o_ref):
    deg = jnp.clip(d0_ref[...] + d1_ref[...], 1.0, None)
    agg = (a0_ref[...] + a1_ref[...]) / deg
    out = out_ref[...]
    h = h_ref[...]
    m = jax.nn.relu(
        agg + jnp.dot(out, root_ref[...], preferred_element_type=_f32) + cb_ref[...])
    ir = jnp.dot(m, wir_ref[...], preferred_element_type=_f32) + bir_ref[...]
    iz = jnp.dot(m, wiz_ref[...], preferred_element_type=_f32) + biz_ref[...]
    i_n = jnp.dot(m, win_ref[...], preferred_element_type=_f32) + bin_ref[...]
    hr = jnp.dot(h, whr_ref[...], preferred_element_type=_f32) + bhr_ref[...]
    hz = jnp.dot(h, whz_ref[...], preferred_element_type=_f32) + bhz_ref[...]
    hn = jnp.dot(h, whn_ref[...], preferred_element_type=_f32) + bhn_ref[...]
    r = jax.nn.sigmoid(ir + hr)
    z = jax.nn.sigmoid(iz + hz)
    n = jnp.tanh(i_n + r * hn)
    o_ref[...] = (1.0 - z) * n + z * h


def _set2set_body(out_ref, batch_ref, wq_ref, wr_ref, wh_ref, b_ref,
                  l1q_ref, l1r_ref, l1b_ref, l2_ref, l2b_ref, y_ref):
    out = out_ref[...]
    bvec = batch_ref[...]
    iota = lax.broadcasted_iota(jnp.int32, (N, B), 1)
    maskb = bvec == iota
    mask = maskb.astype(_f32)
    q = jnp.zeros((B, DIM), _f32)
    rv = jnp.zeros((B, DIM), _f32)
    hh = jnp.zeros((B, DIM), _f32)
    cc = jnp.zeros((B, DIM), _f32)
    for _ in range(3):
        g = (jnp.dot(q, wq_ref[...], preferred_element_type=_f32)
             + jnp.dot(rv, wr_ref[...], preferred_element_type=_f32)
             + jnp.dot(hh, wh_ref[...], preferred_element_type=_f32)
             + b_ref[...])
        i_ = jax.nn.sigmoid(g[:, 0:DIM])
        f_ = jax.nn.sigmoid(g[:, DIM:2 * DIM])
        g_ = jnp.tanh(g[:, 2 * DIM:3 * DIM])
        o_ = jax.nn.sigmoid(g[:, 3 * DIM:4 * DIM])
        cc = f_ * cc + i_ * g_
        hh = o_ * jnp.tanh(cc)
        q = hh
        qb = jnp.dot(mask, q, preferred_element_type=_f32)
        e = jnp.sum(out * qb, axis=1, keepdims=True)
        eb = jnp.where(maskb, e, -1e30)
        emax = jnp.max(eb, axis=0, keepdims=True)
        emax = jnp.where(emax < -1e29, 0.0, emax)
        erow = jnp.sum(mask * emax, axis=1, keepdims=True)
        a = jnp.exp(e - erow)
        asum = jnp.sum(a * mask, axis=0, keepdims=True)
        arow = jnp.sum(mask * asum, axis=1, keepdims=True)
        an = a / (arow + 1e-16)
        w = mask * an
        rv = lax.dot_general(w, out, (((0,), (0,)), ((), ())),
                             preferred_element_type=_f32)
    y1 = jax.nn.relu(jnp.dot(q, l1q_ref[...], preferred_element_type=_f32)
                     + jnp.dot(rv, l1r_ref[...], preferred_element_type=_f32)
                     + l1b_ref[...])
    y_ref[...] = jnp.dot(y1, l2_ref[...], preferred_element_type=_f32) + l2b_ref[...]


def _tc_call(body, out_shape, *args):
    return pl.pallas_call(body, out_shape=out_shape)(*args)


def kernel(x, edge_index, edge_attr, batch, lin0_W, lin0_b, nn1_W, nn1_b,
           nn2_W, nn2_b, root_W, conv_bias, gru_Wih, gru_Whh, gru_bih,
           gru_bhh, lstm_Wih, lstm_Whh, lstm_bih, lstm_bhh, lin1_W, lin1_b,
           lin2_W, lin2_b):
    src = edge_index[0].astype(jnp.int32)
    dst = edge_index[1].astype(jnp.int32)
    src2 = jnp.pad(src, (0, EP - E)).reshape(EP // CHUNK, CHUNK)
    dst2 = jnp.pad(dst, (0, EP - E), constant_values=N).reshape(EP // CHUNK, CHUNK)
    ea_p = jnp.pad(edge_attr, ((0, EP - E), (0, 0)))
    zeros_hbm = jnp.zeros((NPAD, DIM), _f32)
    ones_msg = jnp.ones((EP, DIM), _f32)

    out0 = _tc_call(_lin0_body, jax.ShapeDtypeStruct((N, DIM), _f32),
                    x, lin0_W.T, lin0_b.reshape(1, DIM))

    he = pl.pallas_call(
        _he_body,
        grid=(NBLK,),
        in_specs=[
            pl.BlockSpec((BE, 4), lambda i: (i, 0)),
            pl.BlockSpec((4, MESS), lambda i: (0, 0)),
            pl.BlockSpec((1, MESS), lambda i: (0, 0)),
        ],
        out_specs=pl.BlockSpec((BE, MESS), lambda i: (i, 0)),
        out_shape=jax.ShapeDtypeStruct((EP, MESS), _f32),
    )(ea_p, nn1_W.T, nn1_b.reshape(1, MESS))

    degp = _sc_scatter_add(ones_msg, dst2, zeros_hbm)
    d0 = degp[0, :N, 0:1]
    d1 = degp[1, :N, 0:1]

    gWihT = gru_Wih.T
    gWhhT = gru_Whh.T
    upd_w = (root_W, conv_bias.reshape(1, DIM),
             gWihT[:, 0:DIM], gWihT[:, DIM:2 * DIM], gWihT[:, 2 * DIM:3 * DIM],
             gWhhT[:, 0:DIM], gWhhT[:, DIM:2 * DIM], gWhhT[:, 2 * DIM:3 * DIM],
             gru_bih[0:DIM].reshape(1, DIM), gru_bih[DIM:2 * DIM].reshape(1, DIM),
             gru_bih[2 * DIM:].reshape(1, DIM),
             gru_bhh[0:DIM].reshape(1, DIM), gru_bhh[DIM:2 * DIM].reshape(1, DIM),
             gru_bhh[2 * DIM:].reshape(1, DIM))

    out = out0
    h = out0
    nn2T = nn2_W.T
    nn2b = nn2_b.reshape(1, DIM * DIM)
    for _ in range(3):
        xj = _sc_gather(out, src2)
        msg = pl.pallas_call(
            _msg_body,
            grid=(NBLK,),
            in_specs=[
                pl.BlockSpec((BE, DIM), lambda i: (i, 0)),
                pl.BlockSpec((BE, MESS), lambda i: (i, 0)),
                pl.BlockSpec((MESS, DIM * DIM), lambda i: (0, 0)),
                pl.BlockSpec((1, DIM * DIM), lambda i: (0, 0)),
            ],
            out_specs=pl.BlockSpec((BE, DIM), lambda i: (i, 0)),
            out_shape=jax.ShapeDtypeStruct((EP, DIM), _f32),
        )(xj, he, nn2T, nn2b)
        aggp = _sc_scatter_add(msg, dst2, zeros_hbm)
        h = _tc_call(_update_body, jax.ShapeDtypeStruct((N, DIM), _f32),
                     aggp[0, :N], aggp[1, :N], d0, d1, out, h, *upd_w)
        out = h

    lWihT = lstm_Wih.T
    y = _tc_call(
        _set2set_body, jax.ShapeDtypeStruct((B, 1), _f32),
        out, batch.astype(jnp.int32).reshape(N, 1),
        lWihT[0:DIM], lWihT[DIM:2 * DIM], lstm_Whh.T,
        (lstm_bih + lstm_bhh).reshape(1, 4 * DIM),
        lin1_W.T[0:DIM], lin1_W.T[DIM:2 * DIM], lin1_b.reshape(1, DIM),
        lin2_W.T, lin2_b.reshape(1, 1))
    return y.reshape(-1)


# speedup vs baseline: 2.4701x; 2.4701x over previous
import jax
import jax.numpy as jnp
from jax import lax
from jax.experimental import pallas as pl
from jax.experimental.pallas import tpu as pltpu
from jax.experimental.pallas import tpu_sc as plsc

N = 10000
E = 160000
DIN = 128
DIM = 32
MESS = 128
B = 64

EP = 163840
NW = 32
CHUNK = 128
CPW = EP // NW // CHUNK
RPW = EP // NW
NPAD = 10016
RPT = NPAD // 16
BE = 1280
NBLK = EP // BE

_f32 = jnp.float32


def _sc_mesh():
    return plsc.VectorSubcoreMesh(core_axis_name="c", subcore_axis_name="s")


def _gather_body(table, idx2, xj, idx_v, rows_v, sem):
    c = lax.axis_index("c")
    s = lax.axis_index("s")
    wid = s * 2 + c
    pltpu.sync_copy(idx2.at[pl.ds(wid * CPW, CPW)], idx_v)

    def body(g, carry):
        cps = [
            pltpu.async_copy(table.at[idx_v.at[g * 4 + j]],
                             rows_v.at[pl.ds(j * CHUNK, CHUNK)], sem)
            for j in range(4)
        ]
        for cp in cps:
            cp.wait()
        pltpu.sync_copy(rows_v, xj.at[pl.ds(wid * RPW + g * 4 * CHUNK, 4 * CHUNK)])
        return carry

    lax.fori_loop(0, CPW // 4, body, 0)


def _sc_gather(table, idx2):
    return pl.kernel(
        _gather_body,
        out_type=jax.ShapeDtypeStruct((EP, DIM), _f32),
        mesh=_sc_mesh(),
        scratch_types=[
            pltpu.VMEM((CPW, CHUNK), jnp.int32),
            pltpu.VMEM((4 * CHUNK, DIM), _f32),
            pltpu.SemaphoreType.DMA,
        ],
        compiler_params=pltpu.CompilerParams(use_tc_tiling_on_sc=False),
    )(table, idx2)


def _scatter_body(msg, idx2, zeros_hbm, aggp, idx_v, rows_v, agg_sh):
    c = lax.axis_index("c")
    s = lax.axis_index("s")
    wid = s * 2 + c
    pltpu.sync_copy(zeros_hbm.at[pl.ds(s * RPT, RPT)], agg_sh.at[pl.ds(s * RPT, RPT)])
    plsc.subcore_barrier()
    pltpu.sync_copy(idx2.at[pl.ds(wid * CPW, CPW)], idx_v)

    def body(g, carry):
        pltpu.sync_copy(msg.at[pl.ds(wid * RPW + g * CHUNK, CHUNK)], rows_v)
        pltpu.sync_copy(rows_v, agg_sh.at[idx_v.at[g]], add=True)
        return carry

    lax.fori_loop(0, CPW, body, 0)
    plsc.subcore_barrier()
    pltpu.sync_copy(agg_sh.at[pl.ds(s * RPT, RPT)], aggp.at[c, pl.ds(s * RPT, RPT)])


def _sc_scatter_add(msg, idx2, zeros_hbm):
    return pl.kernel(
        _scatter_body,
        out_type=jax.ShapeDtypeStruct((2, NPAD, DIM), _f32),
        mesh=_sc_mesh(),
        scratch_types=[
            pltpu.VMEM((CPW, CHUNK), jnp.int32),
            pltpu.VMEM((CHUNK, DIM), _f32),
            pltpu.VMEM_SHARED((NPAD, DIM), _f32),
        ],
        compiler_params=pltpu.CompilerParams(use_tc_tiling_on_sc=False),
    )(msg, idx2, zeros_hbm)


def _lin0_body(x_ref, w_ref, b_ref, o_ref):
    o_ref[...] = jax.nn.relu(
        jnp.dot(x_ref[...], w_ref[...], preferred_element_type=_f32) + b_ref[...])


def _he_body(ea_ref, w_ref, b_ref, o_ref):
    o_ref[...] = jax.nn.relu(
        jnp.dot(ea_ref[...], w_ref[...], preferred_element_type=_f32) + b_ref[...])


def _msg_body(xj_ref, he_ref, w_ref, b_ref, rep_ref, red_ref, o_ref):
    we = jnp.dot(he_ref[...], w_ref[...], preferred_element_type=_f32) + b_ref[...]
    xjrep = jnp.dot(xj_ref[...], rep_ref[...], preferred_element_type=_f32)
    o_ref[...] = jnp.dot(we * xjrep, red_ref[...], preferred_element_type=_f32)


def _update_body(a0_ref, a1_ref, d0_ref, d1_ref, out_ref, h_ref, root_ref,
                 cb_ref, wir_ref, wiz_ref, win_ref, whr_ref, whz_ref, whn_ref,
                 bir_ref, biz_ref, bin_ref, bhr_ref, bhz_ref, bhn_ref, o_ref):
    deg = jnp.clip(d0_ref[...] + d1_ref[...], 1.0, None)
    agg = (a0_ref[...] + a1_ref[...]) / deg
    out = out_ref[...]
    h = h_ref[...]
    m = jax.nn.relu(
        agg + jnp.dot(out, root_ref[...], preferred_element_type=_f32) + cb_ref[...])
    ir = jnp.dot(m, wir_ref[...], preferred_element_type=_f32) + bir_ref[...]
    iz = jnp.dot(m, wiz_ref[...], preferred_element_type=_f32) + biz_ref[...]
    i_n = jnp.dot(m, win_ref[...], preferred_element_type=_f32) + bin_ref[...]
    hr = jnp.dot(h, whr_ref[...], preferred_element_type=_f32) + bhr_ref[...]
    hz = jnp.dot(h, whz_ref[...], preferred_element_type=_f32) + bhz_ref[...]
    hn = jnp.dot(h, whn_ref[...], preferred_element_type=_f32) + bhn_ref[...]
    r = jax.nn.sigmoid(ir + hr)
    z = jax.nn.sigmoid(iz + hz)
    n = jnp.tanh(i_n + r * hn)
    o_ref[...] = (1.0 - z) * n + z * h


def _set2set_body(out_ref, batch_ref, wq_ref, wr_ref, wh_ref, b_ref,
                  l1q_ref, l1r_ref, l1b_ref, l2_ref, l2b_ref, y_ref):
    out = out_ref[...]
    bvec = batch_ref[...]
    iota = lax.broadcasted_iota(jnp.int32, (N, B), 1)
    maskb = bvec == iota
    mask = maskb.astype(_f32)
    q = jnp.zeros((B, DIM), _f32)
    rv = jnp.zeros((B, DIM), _f32)
    hh = jnp.zeros((B, DIM), _f32)
    cc = jnp.zeros((B, DIM), _f32)
    for _ in range(3):
        g = (jnp.dot(q, wq_ref[...], preferred_element_type=_f32)
             + jnp.dot(rv, wr_ref[...], preferred_element_type=_f32)
             + jnp.dot(hh, wh_ref[...], preferred_element_type=_f32)
             + b_ref[...])
        i_ = jax.nn.sigmoid(g[:, 0:DIM])
        f_ = jax.nn.sigmoid(g[:, DIM:2 * DIM])
        g_ = jnp.tanh(g[:, 2 * DIM:3 * DIM])
        o_ = jax.nn.sigmoid(g[:, 3 * DIM:4 * DIM])
        cc = f_ * cc + i_ * g_
        hh = o_ * jnp.tanh(cc)
        q = hh
        qb = jnp.dot(mask, q, preferred_element_type=_f32)
        e = jnp.sum(out * qb, axis=1, keepdims=True)
        eb = jnp.where(maskb, e, -1e30)
        emax = jnp.max(eb, axis=0, keepdims=True)
        emax = jnp.where(emax < -1e29, 0.0, emax)
        erow = jnp.sum(mask * emax, axis=1, keepdims=True)
        a = jnp.exp(e - erow)
        asum = jnp.sum(a * mask, axis=0, keepdims=True)
        arow = jnp.sum(mask * asum, axis=1, keepdims=True)
        an = a / (arow + 1e-16)
        w = mask * an
        rv = lax.dot_general(w, out, (((0,), (0,)), ((), ())),
                             preferred_element_type=_f32)
    y1 = jax.nn.relu(jnp.dot(q, l1q_ref[...], preferred_element_type=_f32)
                     + jnp.dot(rv, l1r_ref[...], preferred_element_type=_f32)
                     + l1b_ref[...])
    y_ref[...] = jnp.dot(y1, l2_ref[...], preferred_element_type=_f32) + l2b_ref[...]


def _tc_call(body, out_shape, *args):
    return pl.pallas_call(body, out_shape=out_shape)(*args)


def kernel(x, edge_index, edge_attr, batch, lin0_W, lin0_b, nn1_W, nn1_b,
           nn2_W, nn2_b, root_W, conv_bias, gru_Wih, gru_Whh, gru_bih,
           gru_bhh, lstm_Wih, lstm_Whh, lstm_bih, lstm_bhh, lin1_W, lin1_b,
           lin2_W, lin2_b):
    src = edge_index[0].astype(jnp.int32)
    dst = edge_index[1].astype(jnp.int32)
    src2 = jnp.pad(src, (0, EP - E)).reshape(EP // CHUNK, CHUNK)
    dst2 = jnp.pad(dst, (0, EP - E), constant_values=N).reshape(EP // CHUNK, CHUNK)
    ea_p = jnp.pad(edge_attr, ((0, EP - E), (0, 0)))
    zeros_hbm = jnp.zeros((NPAD, DIM), _f32)
    ones_msg = jnp.ones((EP, DIM), _f32)

    out0 = _tc_call(_lin0_body, jax.ShapeDtypeStruct((N, DIM), _f32),
                    x, lin0_W.T, lin0_b.reshape(1, DIM))

    he = pl.pallas_call(
        _he_body,
        grid=(NBLK,),
        in_specs=[
            pl.BlockSpec((BE, 4), lambda i: (i, 0)),
            pl.BlockSpec((4, MESS), lambda i: (0, 0)),
            pl.BlockSpec((1, MESS), lambda i: (0, 0)),
        ],
        out_specs=pl.BlockSpec((BE, MESS), lambda i: (i, 0)),
        out_shape=jax.ShapeDtypeStruct((EP, MESS), _f32),
    )(ea_p, nn1_W.T, nn1_b.reshape(1, MESS))

    degp = _sc_scatter_add(ones_msg, dst2, zeros_hbm)
    d0 = degp[0, :N, 0:1]
    d1 = degp[1, :N, 0:1]

    gWihT = gru_Wih.T
    gWhhT = gru_Whh.T
    upd_w = (root_W, conv_bias.reshape(1, DIM),
             gWihT[:, 0:DIM], gWihT[:, DIM:2 * DIM], gWihT[:, 2 * DIM:3 * DIM],
             gWhhT[:, 0:DIM], gWhhT[:, DIM:2 * DIM], gWhhT[:, 2 * DIM:3 * DIM],
             gru_bih[0:DIM].reshape(1, DIM), gru_bih[DIM:2 * DIM].reshape(1, DIM),
             gru_bih[2 * DIM:].reshape(1, DIM),
             gru_bhh[0:DIM].reshape(1, DIM), gru_bhh[DIM:2 * DIM].reshape(1, DIM),
             gru_bhh[2 * DIM:].reshape(1, DIM))

    out = out0
    h = out0
    nn2T = nn2_W.T
    nn2b = nn2_b.reshape(1, DIM * DIM)
    col = jnp.arange(DIM * DIM, dtype=jnp.int32)
    rep_m = (col[None, :] // DIM == jnp.arange(DIM, dtype=jnp.int32)[:, None]
             ).astype(_f32)
    red_m = (col[:, None] % DIM == jnp.arange(DIM, dtype=jnp.int32)[None, :]
             ).astype(_f32)
    for _ in range(3):
        xj = _sc_gather(out, src2)
        msg = pl.pallas_call(
            _msg_body,
            grid=(NBLK,),
            in_specs=[
                pl.BlockSpec((BE, DIM), lambda i: (i, 0)),
                pl.BlockSpec((BE, MESS), lambda i: (i, 0)),
                pl.BlockSpec((MESS, DIM * DIM), lambda i: (0, 0)),
                pl.BlockSpec((1, DIM * DIM), lambda i: (0, 0)),
                pl.BlockSpec((DIM, DIM * DIM), lambda i: (0, 0)),
                pl.BlockSpec((DIM * DIM, DIM), lambda i: (0, 0)),
            ],
            out_specs=pl.BlockSpec((BE, DIM), lambda i: (i, 0)),
            out_shape=jax.ShapeDtypeStruct((EP, DIM), _f32),
        )(xj, he, nn2T, nn2b, rep_m, red_m)
        aggp = _sc_scatter_add(msg, dst2, zeros_hbm)
        h = _tc_call(_update_body, jax.ShapeDtypeStruct((N, DIM), _f32),
                     aggp[0, :N], aggp[1, :N], d0, d1, out, h, *upd_w)
        out = h

    lWihT = lstm_Wih.T
    y = _tc_call(
        _set2set_body, jax.ShapeDtypeStruct((B, 1), _f32),
        out, batch.astype(jnp.int32).reshape(N, 1),
        lWihT[0:DIM], lWihT[DIM:2 * DIM], lstm_Whh.T,
        (lstm_bih + lstm_bhh).reshape(1, 4 * DIM),
        lin1_W.T[0:DIM], lin1_W.T[DIM:2 * DIM], lin1_b.reshape(1, DIM),
        lin2_W.T, lin2_b.reshape(1, 1))
    return y.reshape(-1)
